# baseline (speedup 1.0000x reference)
import functools

import jax
import jax.numpy as jnp
from jax.experimental import pallas as pl
from jax.experimental.pallas import tpu as pltpu

_D = 1024
_F = 4096
_E = 5
_L = 4
_R = 16
_SCALE = 2.0
_AUXW = 0.01

_TILE_T = 1024
_TILE_F = 1024


def _moe_body(x_ref, wr_ref, w1_ref, b1_ref, w2_ref, b2_ref, acat_ref,
              bcat_ref, out_ref, aux_ref, stat_ref, *, n_tokens):
    i = pl.program_id(0)
    j = pl.program_id(1)
    ni = pl.num_programs(0)
    nj = pl.num_programs(1)
    x = x_ref[...]

    h = jax.lax.dot_general(
        x, w1_ref[...], (((1,), (1,)), ((), ())),
        preferred_element_type=jnp.float32) + b1_ref[...]
    h = 0.5 * h * (1.0 + jax.lax.erf(h * 0.7071067811865476))
    base = jax.lax.dot_general(
        h, w2_ref[...], (((1,), (1,)), ((), ())),
        preferred_element_type=jnp.float32)

    @pl.when(j == 0)
    def _init_out():
        out_ref[...] = base + b2_ref[...]

    @pl.when(j > 0)
    def _acc_out():
        out_ref[...] += base

    @pl.when(j == nj - 1)
    def _router_and_lora():
        logits = jax.lax.dot_general(
            x, wr_ref[...], (((1,), (1,)), ((), ())),
            preferred_element_type=jnp.float32)
        sel = jnp.argmax(logits, axis=1, keepdims=True).astype(jnp.int32)
        t_all = jax.lax.dot_general(
            x, acat_ref[...], (((1,), (1,)), ((), ())),
            preferred_element_type=jnp.float32)
        col_e = jax.lax.broadcasted_iota(jnp.int32, t_all.shape, 1) // _R + 1
        t_m = jnp.where(sel == col_e, t_all, 0.0)
        lora = jax.lax.dot_general(
            t_m, bcat_ref[...], (((1,), (0,)), ((), ())),
            preferred_element_type=jnp.float32) * _SCALE
        out_ref[...] += lora

        probs = jax.nn.softmax(logits, axis=-1)
        eidx = jax.lax.broadcasted_iota(jnp.int32, logits.shape, 1)
        counts = (sel == eidx).astype(jnp.float32)

        @pl.when(i == 0)
        def _zero_stats():
            stat_ref[...] = jnp.zeros_like(stat_ref)

        stat_ref[0:1, :_E] += jnp.sum(probs, axis=0, keepdims=True)
        stat_ref[1:2, :_E] += jnp.sum(counts, axis=0, keepdims=True)

        @pl.when(i == ni - 1)
        def _finish_aux():
            pm = stat_ref[0:1, :_E] * (1.0 / n_tokens)
            cm = stat_ref[1:2, :_E] * (1.0 / n_tokens)
            aux_ref[...] = jnp.sum(pm * cm, keepdims=True) * (_E * _AUXW)


def kernel(hidden_states, Wr, W1, b1, W2, b2, A, B):
    bsz, seq, d = hidden_states.shape
    n = bsz * seq
    x2 = hidden_states.reshape(n, d)
    a_cat = A.reshape(_L * _R, _D)
    b_cat = B.transpose(0, 2, 1).reshape(_L * _R, _D)
    b1r = b1.reshape(1, _F)
    b2r = b2.reshape(1, _D)

    grid = (n // _TILE_T, _F // _TILE_F)
    out, aux = pl.pallas_call(
        functools.partial(_moe_body, n_tokens=n),
        grid=grid,
        in_specs=[
            pl.BlockSpec((_TILE_T, _D), lambda i, j: (i, 0)),
            pl.BlockSpec((_E, _D), lambda i, j: (0, 0)),
            pl.BlockSpec((_TILE_F, _D), lambda i, j: (j, 0)),
            pl.BlockSpec((1, _TILE_F), lambda i, j: (0, j)),
            pl.BlockSpec((_D, _TILE_F), lambda i, j: (0, j)),
            pl.BlockSpec((1, _D), lambda i, j: (0, 0)),
            pl.BlockSpec((_L * _R, _D), lambda i, j: (0, 0)),
            pl.BlockSpec((_L * _R, _D), lambda i, j: (0, 0)),
        ],
        out_specs=[
            pl.BlockSpec((_TILE_T, _D), lambda i, j: (i, 0)),
            pl.BlockSpec((1, 1), lambda i, j: (0, 0)),
        ],
        out_shape=[
            jax.ShapeDtypeStruct((n, _D), jnp.float32),
            jax.ShapeDtypeStruct((1, 1), jnp.float32),
        ],
        scratch_shapes=[pltpu.VMEM((8, 128), jnp.float32)],
        compiler_params=pltpu.CompilerParams(
            dimension_semantics=("arbitrary", "arbitrary")),
    )(x2, Wr, W1, b1r, W2, b2r, a_cat, b_cat)
    return out.reshape(bsz, seq, d), aux[0, 0]


# speedup vs baseline: 1.0206x; 1.0206x over previous
import functools

import jax
import jax.numpy as jnp
from jax.experimental import pallas as pl
from jax.experimental.pallas import tpu as pltpu

_D = 1024
_F = 4096
_E = 5
_L = 4
_R = 16
_SCALE = 2.0
_AUXW = 0.01

_TILE_T = 1024
_TILE_F = 1024


def _moe_body(x_ref, wr_ref, w1_ref, b1_ref, w2_ref, b2_ref, acat_ref,
              bcat_ref, out_ref, aux_ref, stat_ref, *, n_tokens):
    i = pl.program_id(0)
    j = pl.program_id(1)
    ni = pl.num_programs(0)
    nj = pl.num_programs(1)
    x = x_ref[...]

    @pl.when(j == 0)
    def _router_and_lora():
        xr = x_ref[...]
        logits = jax.lax.dot_general(
            xr, wr_ref[...], (((1,), (1,)), ((), ())),
            preferred_element_type=jnp.float32)
        sel = jnp.argmax(logits, axis=1, keepdims=True).astype(jnp.int32)
        t_all = jax.lax.dot_general(
            xr, acat_ref[...], (((1,), (1,)), ((), ())),
            preferred_element_type=jnp.float32)
        col_e = jax.lax.broadcasted_iota(jnp.int32, t_all.shape, 1) // _R + 1
        t_m = jnp.where(sel == col_e, t_all, 0.0)
        lora = jax.lax.dot_general(
            t_m, bcat_ref[...], (((1,), (0,)), ((), ())),
            preferred_element_type=jnp.float32) * _SCALE
        out_ref[...] = lora + b2_ref[...]

        probs = jax.nn.softmax(logits, axis=-1)
        eidx = jax.lax.broadcasted_iota(jnp.int32, logits.shape, 1)
        counts = (sel == eidx).astype(jnp.float32)

        @pl.when(i == 0)
        def _zero_stats():
            stat_ref[...] = jnp.zeros_like(stat_ref)

        stat_ref[0:1, :_E] += jnp.sum(probs, axis=0, keepdims=True)
        stat_ref[1:2, :_E] += jnp.sum(counts, axis=0, keepdims=True)

    h = jax.lax.dot_general(
        x, w1_ref[...], (((1,), (1,)), ((), ())),
        preferred_element_type=jnp.float32) + b1_ref[...]
    h = 0.5 * h * (1.0 + jax.lax.erf(h * 0.7071067811865476))
    out_ref[...] += jax.lax.dot_general(
        h, w2_ref[...], (((1,), (1,)), ((), ())),
        preferred_element_type=jnp.float32)

    @pl.when((i == ni - 1) & (j == nj - 1))
    def _finish_aux():
        pm = stat_ref[0:1, :_E] * (1.0 / n_tokens)
        cm = stat_ref[1:2, :_E] * (1.0 / n_tokens)
        aux_ref[...] = jnp.sum(pm * cm, keepdims=True) * (_E * _AUXW)


def kernel(hidden_states, Wr, W1, b1, W2, b2, A, B):
    bsz, seq, d = hidden_states.shape
    n = bsz * seq
    x2 = hidden_states.reshape(n, d)
    a_cat = A.reshape(_L * _R, _D)
    b_cat = B.transpose(0, 2, 1).reshape(_L * _R, _D)
    b1r = b1.reshape(1, _F)
    b2r = b2.reshape(1, _D)

    grid = (n // _TILE_T, _F // _TILE_F)
    out, aux = pl.pallas_call(
        functools.partial(_moe_body, n_tokens=n),
        grid=grid,
        in_specs=[
            pl.BlockSpec((_TILE_T, _D), lambda i, j: (i, 0)),
            pl.BlockSpec((_E, _D), lambda i, j: (0, 0)),
            pl.BlockSpec((_TILE_F, _D), lambda i, j: (j, 0)),
            pl.BlockSpec((1, _TILE_F), lambda i, j: (0, j)),
            pl.BlockSpec((_D, _TILE_F), lambda i, j: (0, j)),
            pl.BlockSpec((1, _D), lambda i, j: (0, 0)),
            pl.BlockSpec((_L * _R, _D), lambda i, j: (0, 0)),
            pl.BlockSpec((_L * _R, _D), lambda i, j: (0, 0)),
        ],
        out_specs=[
            pl.BlockSpec((_TILE_T, _D), lambda i, j: (i, 0)),
            pl.BlockSpec((1, 1), lambda i, j: (0, 0)),
        ],
        out_shape=[
            jax.ShapeDtypeStruct((n, _D), jnp.float32),
            jax.ShapeDtypeStruct((1, 1), jnp.float32),
        ],
        scratch_shapes=[pltpu.VMEM((8, 128), jnp.float32)],
        compiler_params=pltpu.CompilerParams(
            dimension_semantics=("arbitrary", "arbitrary")),
    )(x2, Wr, W1, b1r, W2, b2r, a_cat, b_cat)
    return out.reshape(bsz, seq, d), aux[0, 0]


# speedup vs baseline: 1.0435x; 1.0225x over previous
import functools

import jax
import jax.numpy as jnp
from jax.experimental import pallas as pl
from jax.experimental.pallas import tpu as pltpu

_D = 1024
_F = 4096
_E = 5
_L = 4
_R = 16
_SCALE = 2.0
_AUXW = 0.01

_TILE_T = 512


def _moe_body(x_ref, wr_ref, w1_ref, b1_ref, w2_ref, b2_ref, acat_ref,
              bcat_ref, out_ref, aux_ref, stat_ref, *, n_tokens):
    i = pl.program_id(0)
    ni = pl.num_programs(0)
    x = x_ref[...]

    h = jax.lax.dot_general(
        x, w1_ref[...], (((1,), (1,)), ((), ())),
        preferred_element_type=jnp.float32) + b1_ref[...]
    h = 0.5 * h * (1.0 + jax.lax.erf(h * 0.7071067811865476))
    base = jax.lax.dot_general(
        h, w2_ref[...], (((1,), (1,)), ((), ())),
        preferred_element_type=jnp.float32)

    logits = jax.lax.dot_general(
        x, wr_ref[...], (((1,), (1,)), ((), ())),
        preferred_element_type=jnp.float32)
    sel = jnp.argmax(logits, axis=1, keepdims=True).astype(jnp.int32)
    t_all = jax.lax.dot_general(
        x, acat_ref[...], (((1,), (1,)), ((), ())),
        preferred_element_type=jnp.float32)
    col_e = jax.lax.broadcasted_iota(jnp.int32, t_all.shape, 1) // _R + 1
    t_m = jnp.where(sel == col_e, t_all, 0.0)
    lora = jax.lax.dot_general(
        t_m, bcat_ref[...], (((1,), (0,)), ((), ())),
        preferred_element_type=jnp.float32) * _SCALE
    out_ref[...] = base + lora + b2_ref[...]

    probs = jax.nn.softmax(logits, axis=-1)
    eidx = jax.lax.broadcasted_iota(jnp.int32, logits.shape, 1)
    counts = (sel == eidx).astype(jnp.float32)

    @pl.when(i == 0)
    def _zero_stats():
        stat_ref[...] = jnp.zeros_like(stat_ref)

    stat_ref[0:1, :_E] += jnp.sum(probs, axis=0, keepdims=True)
    stat_ref[1:2, :_E] += jnp.sum(counts, axis=0, keepdims=True)

    @pl.when(i == ni - 1)
    def _finish_aux():
        pm = stat_ref[0:1, :_E] * (1.0 / n_tokens)
        cm = stat_ref[1:2, :_E] * (1.0 / n_tokens)
        aux_ref[...] = jnp.sum(pm * cm, keepdims=True) * (_E * _AUXW)


def kernel(hidden_states, Wr, W1, b1, W2, b2, A, B):
    bsz, seq, d = hidden_states.shape
    n = bsz * seq
    x2 = hidden_states.reshape(n, d)
    a_cat = A.reshape(_L * _R, _D)
    b_cat = B.transpose(0, 2, 1).reshape(_L * _R, _D)
    b1r = b1.reshape(1, _F)
    b2r = b2.reshape(1, _D)

    grid = (n // _TILE_T,)
    out, aux = pl.pallas_call(
        functools.partial(_moe_body, n_tokens=n),
        grid=grid,
        in_specs=[
            pl.BlockSpec((_TILE_T, _D), lambda i: (i, 0)),
            pl.BlockSpec((_E, _D), lambda i: (0, 0)),
            pl.BlockSpec((_F, _D), lambda i: (0, 0)),
            pl.BlockSpec((1, _F), lambda i: (0, 0)),
            pl.BlockSpec((_D, _F), lambda i: (0, 0)),
            pl.BlockSpec((1, _D), lambda i: (0, 0)),
            pl.BlockSpec((_L * _R, _D), lambda i: (0, 0)),
            pl.BlockSpec((_L * _R, _D), lambda i: (0, 0)),
        ],
        out_specs=[
            pl.BlockSpec((_TILE_T, _D), lambda i: (i, 0)),
            pl.BlockSpec((1, 1), lambda i: (0, 0)),
        ],
        out_shape=[
            jax.ShapeDtypeStruct((n, _D), jnp.float32),
            jax.ShapeDtypeStruct((1, 1), jnp.float32),
        ],
        scratch_shapes=[pltpu.VMEM((8, 128), jnp.float32)],
        compiler_params=pltpu.CompilerParams(
            dimension_semantics=("arbitrary",)),
    )(x2, Wr, W1, b1r, W2, b2r, a_cat, b_cat)
    return out.reshape(bsz, seq, d), aux[0, 0]
